# baseline (speedup 1.0000x reference)
import functools

import jax
import jax.numpy as jnp
from jax import lax
from jax.experimental import pallas as pl
from jax.experimental.pallas import tpu as pltpu
from jax.experimental.pallas import tpu_sc as plsc

_ENTITY_SIZE = 1000000
_EMBED_DIM = 64
_MARGIN = 1.0

_L = 16
_CHUNK = 128
_NG = _CHUNK // _L

_EPAD = _ENTITY_SIZE + 64
_NTILE = _EPAD // 128
_TROW = _NTILE * 1024
_FLAT = _EMBED_DIM * _EPAD


def _rsqrt_nr(x):
    i = lax.bitcast_convert_type(x, jnp.int32)
    y = lax.bitcast_convert_type(jnp.int32(0x5F3759DF) - (i >> 1), jnp.float32)
    for _ in range(3):
        y = y * (1.5 - 0.5 * x * y * y)
    return y


def _phys_view(table):
    return (jnp.pad(table, ((0, _EPAD - table.shape[0]), (0, 0))).T
            .reshape(8, 8, _NTILE, 128)
            .transpose(0, 2, 1, 3)
            .reshape(_FLAT))


def _gen_gather(idx_refs, gi_refs, tab_refs, val_refs, sem):
    def genf_body(f, carry):
        fc = (f >> 3) * _TROW + (f & 7) * 128

        def geng_body(g, carry2):
            gb = g * _L
            for src_v, dst_v in zip(idx_refs, gi_refs):
                e = src_v[pl.ds(gb, _L)]
                base = ((e >> 7) << 10) + (e & 127)
                dst_v[f, pl.ds(gb, _L)] = base + fc
            return carry2

        lax.fori_loop(0, _NG, geng_body, 0)
        for tab_h, gi_v, val_v in zip(tab_refs, gi_refs, val_refs):
            pltpu.async_copy(tab_h.at[gi_v.at[f]], val_v.at[f], sem)
        return carry

    lax.fori_loop(0, _EMBED_DIM, genf_body, 0)

    def drain_body(f, carry):
        for tab_h, gi_v, val_v in zip(tab_refs, gi_refs, val_refs):
            pltpu.make_async_copy(tab_h.at[gi_v.at[f]], val_v.at[f],
                                  sem).wait()
        return carry

    lax.fori_loop(0, _EMBED_DIM, drain_body, 0)


def _entity_stage(hidx_all, tidx_all, ent_lin, batch):
    info = plsc.get_sparse_core_info()
    nw = info.num_cores * info.num_subcores
    per_w = batch // nw
    n_chunks = per_w // _CHUNK
    total_chunks = 2 * batch // _CHUNK
    mesh = plsc.VectorSubcoreMesh(core_axis_name="c", subcore_axis_name="s")

    @functools.partial(
        pl.kernel,
        mesh=mesh,
        out_type=(
            jax.ShapeDtypeStruct((total_chunks, _EMBED_DIM, _CHUNK),
                                 jnp.float32),
            jax.ShapeDtypeStruct((total_chunks, _CHUNK), jnp.float32),
        ),
        scratch_types=[
            pltpu.VMEM((_CHUNK,), jnp.int32),
            pltpu.VMEM((_CHUNK,), jnp.int32),
            pltpu.VMEM((_EMBED_DIM, _CHUNK), jnp.int32),
            pltpu.VMEM((_EMBED_DIM, _CHUNK), jnp.int32),
            pltpu.VMEM((_EMBED_DIM, _CHUNK), jnp.float32),
            pltpu.VMEM((_EMBED_DIM, _CHUNK), jnp.float32),
            pltpu.VMEM((_EMBED_DIM, _CHUNK), jnp.float32),
            pltpu.VMEM((_CHUNK,), jnp.float32),
            pltpu.SemaphoreType.DMA,
        ],
    )
    def k1(hidx_h, tidx_h, ent_h, u_h, uu_h,
           hidx_v, tidx_v, hgi_v, tgi_v, hval_v, tval_v, u_v, uu_v, sem):
        wid = lax.axis_index("s") * info.num_cores + lax.axis_index("c")
        wbase = wid * per_w

        def chunk_body(c, carry):
            p = c // n_chunks
            cc = c - p * n_chunks
            src = p * batch + wbase + cc * _CHUNK
            gcid = src // _CHUNK
            pltpu.sync_copy(hidx_h.at[pl.ds(src, _CHUNK)], hidx_v)
            pltpu.sync_copy(tidx_h.at[pl.ds(src, _CHUNK)], tidx_v)
            _gen_gather((hidx_v, tidx_v), (hgi_v, tgi_v),
                        (ent_h, ent_h), (hval_v, tval_v), sem)

            def group_body(g, carry2):
                gb = g * _L
                zl = jnp.zeros((_L,), jnp.float32)

                def nrm_body(f, accs):
                    hh, tt = accs
                    h = hval_v[f, pl.ds(gb, _L)]
                    t = tval_v[f, pl.ds(gb, _L)]
                    return (hh + h * h, tt + t * t)

                hh, tt = lax.fori_loop(0, _EMBED_DIM, nrm_body, (zl, zl))

                last = jnp.full((_L,), _ENTITY_SIZE - 1, jnp.int32)
                onef = jnp.ones((_L,), jnp.float32)
                eh = jnp.where(hidx_v[pl.ds(gb, _L)] == last, onef, zl)
                et = jnp.where(tidx_v[pl.ds(gb, _L)] == last, onef, zl)
                a = _rsqrt_nr(hh)
                a = a + eh * (1.0 - a)
                b = _rsqrt_nr(tt)
                b = b + et * (1.0 - b)

                def u_body(f, uu):
                    h = hval_v[f, pl.ds(gb, _L)]
                    t = tval_v[f, pl.ds(gb, _L)]
                    u = h * a - t * b
                    u_v[f, pl.ds(gb, _L)] = u
                    return uu + u * u

                uu = lax.fori_loop(0, _EMBED_DIM, u_body, zl)
                uu_v[pl.ds(gb, _L)] = uu
                return carry2

            lax.fori_loop(0, _NG, group_body, 0)
            pltpu.sync_copy(u_v, u_h.at[gcid])
            pltpu.sync_copy(uu_v, uu_h.at[gcid])
            return carry

        lax.fori_loop(0, 2 * n_chunks, chunk_body, 0)

    return k1(hidx_all, tidx_all, ent_lin)


def _relation_stage(ridx_all, rel_lin, u_all, uu_all, batch):
    info = plsc.get_sparse_core_info()
    nw = info.num_cores * info.num_subcores
    per_w = batch // nw
    n_chunks = per_w // _CHUNK
    mesh = plsc.VectorSubcoreMesh(core_axis_name="c", subcore_axis_name="s")

    @functools.partial(
        pl.kernel,
        mesh=mesh,
        out_type=jax.ShapeDtypeStruct((batch,), jnp.float32),
        scratch_types=[
            pltpu.VMEM((_CHUNK,), jnp.int32),
            pltpu.VMEM((_EMBED_DIM, _CHUNK), jnp.int32),
            pltpu.VMEM((_EMBED_DIM, _CHUNK), jnp.float32),
            pltpu.VMEM((_EMBED_DIM, _CHUNK), jnp.float32),
            pltpu.VMEM((_CHUNK,), jnp.float32),
            pltpu.VMEM((2 * per_w,), jnp.float32),
            pltpu.VMEM((per_w,), jnp.float32),
            pltpu.SemaphoreType.DMA,
        ],
    )
    def k2(ridx_h, rel_h, u_h, uu_h, out_h,
           ridx_v, rgi_v, rval_v, u_v, uu_v, dist_v, loss_v, sem):
        wid = lax.axis_index("s") * info.num_cores + lax.axis_index("c")
        wbase = wid * per_w

        def chunk_body(c, carry):
            p = c // n_chunks
            cc = c - p * n_chunks
            src = p * batch + wbase + cc * _CHUNK
            gcid = src // _CHUNK
            pltpu.sync_copy(ridx_h.at[pl.ds(src, _CHUNK)], ridx_v)
            cp_u = pltpu.async_copy(u_h.at[gcid], u_v, sem)
            cp_uu = pltpu.async_copy(uu_h.at[gcid], uu_v, sem)
            _gen_gather((ridx_v,), (rgi_v,), (rel_h,), (rval_v,), sem)
            cp_u.wait()
            cp_uu.wait()

            dbase = p * per_w + cc * _CHUNK

            def group_body(g, carry2):
                gb = g * _L
                zl = jnp.zeros((_L,), jnp.float32)

                def acc_body(f, accs):
                    rr, ru = accs
                    r = rval_v[f, pl.ds(gb, _L)]
                    u = u_v[f, pl.ds(gb, _L)]
                    return (rr + r * r, ru + r * u)

                rr, ru = lax.fori_loop(0, _EMBED_DIM, acc_body, (zl, zl))
                d2 = uu_v[pl.ds(gb, _L)] + rr + 2.0 * ru
                d2 = jnp.maximum(d2, 0.0)
                d = jnp.where(d2 > 0.0, d2 * _rsqrt_nr(d2), zl)
                dist_v[pl.ds(dbase + gb, _L)] = d
                return carry2

            lax.fori_loop(0, _NG, group_body, 0)
            return carry

        lax.fori_loop(0, 2 * n_chunks, chunk_body, 0)

        def loss_body(g, carry):
            gb = g * _L
            dp = dist_v[pl.ds(gb, _L)]
            dn = dist_v[pl.ds(per_w + gb, _L)]
            loss_v[pl.ds(gb, _L)] = jnp.maximum(dp - dn + _MARGIN, 0.0)
            return carry

        lax.fori_loop(0, per_w // _L, loss_body, 0)
        pltpu.sync_copy(loss_v, out_h.at[pl.ds(wbase, per_w)])

    return k2(ridx_all, rel_lin, u_all, uu_all)


def kernel(positive_triplets, negative_triplets, entity_emb, relation_emb):
    batch = positive_triplets.shape[0]
    cols = jnp.concatenate(
        [positive_triplets.astype(jnp.int32),
         negative_triplets.astype(jnp.int32)], axis=0).T
    u_all, uu_all = _entity_stage(cols[0], cols[2],
                                  _phys_view(entity_emb), batch)
    return _relation_stage(cols[1], _phys_view(relation_emb),
                           u_all, uu_all, batch)


# speedup vs baseline: 1.0933x; 1.0933x over previous
import functools

import jax
import jax.numpy as jnp
from jax import lax
from jax.experimental import pallas as pl
from jax.experimental.pallas import tpu as pltpu
from jax.experimental.pallas import tpu_sc as plsc

_ENTITY_SIZE = 1000000
_EMBED_DIM = 64
_MARGIN = 1.0

_L = 16
_CHUNK = 128
_NG = _CHUNK // _L

_EPAD = _ENTITY_SIZE + 64
_NTILE = _EPAD // 128
_TROW = _NTILE * 1024
_FLAT = _EMBED_DIM * _EPAD


def _rsqrt_nr(x):
    i = lax.bitcast_convert_type(x, jnp.int32)
    y = lax.bitcast_convert_type(jnp.int32(0x5F3759DF) - (i >> 1), jnp.float32)
    for _ in range(3):
        y = y * (1.5 - 0.5 * x * y * y)
    return y


def _phys_view(table):
    return (jnp.pad(table, ((0, _EPAD - table.shape[0]), (0, 0))).T
            .reshape(8, 8, _NTILE, 128)
            .transpose(0, 2, 1, 3)
            .reshape(_FLAT))


def _transe_sc(hidx_all, ridx_all, tidx_all, ent_lin, rel_lin, batch):
    info = plsc.get_sparse_core_info()
    nw = info.num_cores * info.num_subcores
    per_w = batch // nw
    n_chunks = per_w // _CHUNK
    mesh = plsc.VectorSubcoreMesh(core_axis_name="c", subcore_axis_name="s")

    idx_buf = lambda: pltpu.VMEM((_CHUNK,), jnp.int32)
    gi_buf = lambda: pltpu.VMEM((_EMBED_DIM, _CHUNK), jnp.int32)
    val_buf = lambda: pltpu.VMEM((_EMBED_DIM, _CHUNK), jnp.float32)

    @functools.partial(
        pl.kernel,
        mesh=mesh,
        out_type=jax.ShapeDtypeStruct((batch,), jnp.float32),
        scratch_types=[
            [idx_buf(), idx_buf()],
            [idx_buf(), idx_buf()],
            [idx_buf(), idx_buf()],
            [gi_buf(), gi_buf()],
            [gi_buf(), gi_buf()],
            [gi_buf(), gi_buf()],
            [val_buf(), val_buf()],
            [val_buf(), val_buf()],
            [val_buf(), val_buf()],
            pltpu.VMEM((2 * per_w,), jnp.float32),
            pltpu.VMEM((per_w,), jnp.float32),
            [pltpu.SemaphoreType.DMA, pltpu.SemaphoreType.DMA],
        ],
    )
    def k(hidx_h, ridx_h, tidx_h, ent_h, rel_h, out_h,
          hidx_v, ridx_v, tidx_v, hgi_v, rgi_v, tgi_v,
          hval_v, rval_v, tval_v, dist_v, loss_v, sem):
        wid = lax.axis_index("s") * info.num_cores + lax.axis_index("c")
        wbase = wid * per_w

        def launch(c, buf):
            p = c // n_chunks
            cc = c - p * n_chunks
            src = p * batch + wbase + cc * _CHUNK
            pltpu.sync_copy(hidx_h.at[pl.ds(src, _CHUNK)], hidx_v[buf])
            pltpu.sync_copy(ridx_h.at[pl.ds(src, _CHUNK)], ridx_v[buf])
            pltpu.sync_copy(tidx_h.at[pl.ds(src, _CHUNK)], tidx_v[buf])

            def genf_body(f, carry):
                fc = (f >> 3) * _TROW + (f & 7) * 128

                def geng_body(g, carry2):
                    gb = g * _L
                    for src_v, dst_v in ((hidx_v, hgi_v), (ridx_v, rgi_v),
                                         (tidx_v, tgi_v)):
                        e = src_v[buf][pl.ds(gb, _L)]
                        base = ((e >> 7) << 10) + (e & 127)
                        dst_v[buf][f, pl.ds(gb, _L)] = base + fc
                    return carry2

                lax.fori_loop(0, _NG, geng_body, 0)
                pltpu.async_copy(ent_h.at[hgi_v[buf].at[f]],
                                 hval_v[buf].at[f], sem[buf])
                pltpu.async_copy(rel_h.at[rgi_v[buf].at[f]],
                                 rval_v[buf].at[f], sem[buf])
                pltpu.async_copy(ent_h.at[tgi_v[buf].at[f]],
                                 tval_v[buf].at[f], sem[buf])
                return carry

            lax.fori_loop(0, _EMBED_DIM, genf_body, 0)

        def drain(buf):
            def drain_body(f, carry):
                pltpu.make_async_copy(ent_h.at[hgi_v[buf].at[f]],
                                      hval_v[buf].at[f], sem[buf]).wait()
                pltpu.make_async_copy(rel_h.at[rgi_v[buf].at[f]],
                                      rval_v[buf].at[f], sem[buf]).wait()
                pltpu.make_async_copy(ent_h.at[tgi_v[buf].at[f]],
                                      tval_v[buf].at[f], sem[buf]).wait()
                return carry

            lax.fori_loop(0, _EMBED_DIM, drain_body, 0)

        def compute(c, buf):
            p = c // n_chunks
            cc = c - p * n_chunks
            dbase = p * per_w + cc * _CHUNK

            def group_body(g, carry2):
                gb = g * _L
                zl = jnp.zeros((_L,), jnp.float32)

                def acc_body(f, accs):
                    hh, tt, rr, hr, ht, rt = accs
                    h = hval_v[buf][f, pl.ds(gb, _L)]
                    r = rval_v[buf][f, pl.ds(gb, _L)]
                    t = tval_v[buf][f, pl.ds(gb, _L)]
                    return (hh + h * h, tt + t * t, rr + r * r,
                            hr + h * r, ht + h * t, rt + r * t)

                hh, tt, rr, hr, ht, rt = lax.fori_loop(
                    0, _EMBED_DIM, acc_body, (zl, zl, zl, zl, zl, zl))

                last = jnp.full((_L,), _ENTITY_SIZE - 1, jnp.int32)
                onef = jnp.ones((_L,), jnp.float32)
                eh = jnp.where(hidx_v[buf][pl.ds(gb, _L)] == last, onef, zl)
                et = jnp.where(tidx_v[buf][pl.ds(gb, _L)] == last, onef, zl)
                a = _rsqrt_nr(hh)
                a = a + eh * (1.0 - a)
                b = _rsqrt_nr(tt)
                b = b + et * (1.0 - b)
                d2 = hh * (a * a) + rr + tt * (b * b) \
                    + 2.0 * (a * hr - (a * b) * ht - b * rt)
                d2 = jnp.maximum(d2, 0.0)
                d = jnp.where(d2 > 0.0, d2 * _rsqrt_nr(d2), zl)
                dist_v[pl.ds(dbase + gb, _L)] = d
                return carry2

            lax.fori_loop(0, _NG, group_body, 0)

        launch(0, 0)

        def pair_body(cp, carry):
            for half in range(2):
                c = 2 * cp + half

                @pl.when(c + 1 < 2 * n_chunks)
                def _():
                    launch(c + 1, (half + 1) % 2)

                drain(half)
                compute(c, half)
            return carry

        lax.fori_loop(0, n_chunks, pair_body, 0)

        def loss_body(g, carry):
            gb = g * _L
            dp = dist_v[pl.ds(gb, _L)]
            dn = dist_v[pl.ds(per_w + gb, _L)]
            loss_v[pl.ds(gb, _L)] = jnp.maximum(dp - dn + _MARGIN, 0.0)
            return carry

        lax.fori_loop(0, per_w // _L, loss_body, 0)
        pltpu.sync_copy(loss_v, out_h.at[pl.ds(wbase, per_w)])

    return k(hidx_all, ridx_all, tidx_all, ent_lin, rel_lin)


def kernel(positive_triplets, negative_triplets, entity_emb, relation_emb):
    batch = positive_triplets.shape[0]
    cols = jnp.concatenate(
        [positive_triplets.astype(jnp.int32),
         negative_triplets.astype(jnp.int32)], axis=0).T
    return _transe_sc(cols[0], cols[1], cols[2],
                      _phys_view(entity_emb), _phys_view(relation_emb), batch)
